# Initial kernel scaffold; baseline (speedup 1.0000x reference)
import jax
import jax.numpy as jnp
from jax.experimental import pallas as pl


def kernel(x, edge_index, W_mu, b_mu, W_logstd, b_logstd):
    raise NotImplementedError("write your pallas kernel here")


# speedup vs baseline: 20.4139x; 20.4139x over previous
import functools

import jax
import jax.numpy as jnp
from jax import lax
from jax.experimental import pallas as pl
from jax.experimental.pallas import tpu as pltpu
from jax.experimental.pallas import tpu_sc as plsc

N_NODES = 10000
FEAT = 128
NC = 2
NS = 16
NW = NC * NS
CHUNK = 128
CPW = 79
EPAD = NW * CPW * CHUNK
NPAD = 10240
RPS = NPAD // NS
BLK = 1024
LANES = 16

_mesh = plsc.VectorSubcoreMesh(core_axis_name="c", subcore_axis_name="s")


@functools.partial(
    pl.kernel,
    out_type=jax.ShapeDtypeStruct((NC * NPAD,), jnp.float32),
    mesh=_mesh,
    scratch_types=[
        pltpu.VMEM((CHUNK,), jnp.int32),
        pltpu.VMEM((CHUNK,), jnp.float32),
        pltpu.VMEM((RPS,), jnp.float32),
        pltpu.VMEM_SHARED((NPAD,), jnp.float32),
    ],
)
def _sc_degree(col_hbm, deg_hbm, idx_v, ones_v, zbuf_v, acc_sh):
    cix = lax.axis_index("c")
    sid = lax.axis_index("s")
    wid = sid * NC + cix
    for i in range(CHUNK // LANES):
        ones_v[pl.ds(i * LANES, LANES)] = jnp.ones((LANES,), jnp.float32)
    for i in range(RPS // LANES):
        zbuf_v[pl.ds(i * LANES, LANES)] = jnp.zeros((LANES,), jnp.float32)
    pltpu.sync_copy(zbuf_v, acc_sh.at[pl.ds(sid * RPS, RPS)])
    plsc.subcore_barrier()

    def body(j, carry):
        cid = wid * CPW + j
        pltpu.sync_copy(col_hbm.at[cid], idx_v)
        pltpu.sync_copy(ones_v, acc_sh.at[idx_v], add=True)
        return carry

    lax.fori_loop(0, CPW, body, 0)
    plsc.subcore_barrier()
    pltpu.sync_copy(
        acc_sh.at[pl.ds(sid * RPS, RPS)],
        deg_hbm.at[pl.ds(cix * NPAD + sid * RPS, RPS)],
    )


@functools.partial(
    pl.kernel,
    out_type=jax.ShapeDtypeStruct((NC * NPAD, FEAT), jnp.float32),
    mesh=_mesh,
    scratch_types=[
        pltpu.VMEM((CHUNK,), jnp.int32),
        pltpu.VMEM((CHUNK,), jnp.int32),
        pltpu.VMEM((CHUNK, FEAT), jnp.float32),
        pltpu.VMEM_SHARED((NPAD, FEAT), jnp.float32),
        pltpu.SemaphoreType.DMA,
    ],
)
def _sc_edges(row_hbm, col_hbm, y_hbm, out_hbm, ridx_v, cidx_v, rows_v, acc_sh, sem):
    cix = lax.axis_index("c")
    sid = lax.axis_index("s")
    wid = sid * NC + cix

    zeros16 = jnp.zeros((LANES,), jnp.float32)

    def zrow(r, carry):
        for k in range(FEAT // LANES):
            rows_v[r, pl.ds(k * LANES, LANES)] = zeros16
        return carry

    lax.fori_loop(0, CHUNK, zrow, 0)
    for k in range(RPS // CHUNK):
        pltpu.sync_copy(rows_v, acc_sh.at[pl.ds(sid * RPS + k * CHUNK, CHUNK)])
    plsc.subcore_barrier()

    def body(j, carry):
        cid = wid * CPW + j
        pltpu.sync_copy(row_hbm.at[cid], ridx_v)
        pltpu.sync_copy(col_hbm.at[cid], cidx_v)
        pltpu.async_copy(y_hbm.at[ridx_v], rows_v, sem).wait()
        pltpu.sync_copy(rows_v, acc_sh.at[cidx_v], add=True)
        return carry

    lax.fori_loop(0, CPW, body, 0)
    plsc.subcore_barrier()
    pltpu.sync_copy(
        acc_sh.at[pl.ds(sid * RPS, RPS)],
        out_hbm.at[pl.ds(cix * NPAD + sid * RPS, RPS)],
    )


def _tc_prep_body(deg_ref, x_ref, y_ref):
    deg = deg_ref[0, :] + deg_ref[1, :] + 1.0
    dis = lax.rsqrt(deg)
    y_ref[...] = x_ref[...] * dis[:, None]


def _tc_final_body(deg_ref, s_ref, y_ref, wm_ref, bm_ref, wl_ref, bl_ref,
                   mu_ref, ls_ref):
    deg = deg_ref[0, :] + deg_ref[1, :] + 1.0
    dis = lax.rsqrt(deg)
    agg = (s_ref[0] + s_ref[1] + y_ref[...]) * dis[:, None]
    mu_ref[...] = (
        jnp.dot(agg, wm_ref[...], preferred_element_type=jnp.float32,
                precision=lax.Precision.HIGHEST) + bm_ref[...]
    )
    ls_ref[...] = (
        jnp.dot(agg, wl_ref[...], preferred_element_type=jnp.float32,
                precision=lax.Precision.HIGHEST) + bl_ref[...]
    )


def kernel(x, edge_index, W_mu, b_mu, W_logstd, b_logstd):
    row = edge_index[0].astype(jnp.int32)
    col = edge_index[1].astype(jnp.int32)
    e = row.shape[0]
    row_p = jnp.concatenate(
        [row, jnp.zeros((EPAD - e,), jnp.int32)]).reshape(NW * CPW, CHUNK)
    col_p = jnp.concatenate(
        [col, jnp.full((EPAD - e,), N_NODES, jnp.int32)]).reshape(NW * CPW, CHUNK)
    x_p = jnp.concatenate(
        [x, jnp.zeros((NPAD - N_NODES, FEAT), jnp.float32)])

    deg = _sc_degree(col_p).reshape(NC, NPAD)

    y = pl.pallas_call(
        _tc_prep_body,
        grid=(NPAD // BLK,),
        in_specs=[
            pl.BlockSpec((NC, BLK), lambda i: (0, i)),
            pl.BlockSpec((BLK, FEAT), lambda i: (i, 0)),
        ],
        out_specs=pl.BlockSpec((BLK, FEAT), lambda i: (i, 0)),
        out_shape=jax.ShapeDtypeStruct((NPAD, FEAT), jnp.float32),
    )(deg, x_p)

    s = _sc_edges(row_p, col_p, y).reshape(NC, NPAD, FEAT)

    mu_p, ls_p = pl.pallas_call(
        _tc_final_body,
        grid=(NPAD // BLK,),
        in_specs=[
            pl.BlockSpec((NC, BLK), lambda i: (0, i)),
            pl.BlockSpec((NC, BLK, FEAT), lambda i: (0, i, 0)),
            pl.BlockSpec((BLK, FEAT), lambda i: (i, 0)),
            pl.BlockSpec((FEAT, FEAT), lambda i: (0, 0)),
            pl.BlockSpec((1, FEAT), lambda i: (0, 0)),
            pl.BlockSpec((FEAT, FEAT), lambda i: (0, 0)),
            pl.BlockSpec((1, FEAT), lambda i: (0, 0)),
        ],
        out_specs=[
            pl.BlockSpec((BLK, FEAT), lambda i: (i, 0)),
            pl.BlockSpec((BLK, FEAT), lambda i: (i, 0)),
        ],
        out_shape=[
            jax.ShapeDtypeStruct((NPAD, FEAT), jnp.float32),
            jax.ShapeDtypeStruct((NPAD, FEAT), jnp.float32),
        ],
    )(deg, s, y, W_mu, b_mu.reshape(1, FEAT), W_logstd, b_logstd.reshape(1, FEAT))

    return mu_p[:N_NODES], ls_p[:N_NODES]
